# Initial kernel scaffold; baseline (speedup 1.0000x reference)
import jax
import jax.numpy as jnp
from jax.experimental import pallas as pl


def kernel(logits, cumulative_ps, ongoing_beams):
    raise NotImplementedError("write your pallas kernel here")


# speedup vs baseline: 8.1637x; 8.1637x over previous
import functools

import jax
import jax.numpy as jnp
from jax.experimental import pallas as pl
from jax.experimental.pallas import tpu as pltpu

_CHUNK = 8192
_NEG_INF = float("-inf")


def _top8(v, idx):
    big = jnp.int32(2147483647)
    tvs, tis = [], []
    for _ in range(8):
        m = jnp.max(v, axis=1, keepdims=True)
        i = jnp.min(jnp.where(v == m, idx, big), axis=1, keepdims=True)
        tvs.append(m)
        tis.append(i)
        v = jnp.where(idx == i, _NEG_INF, v)
    return jnp.concatenate(tvs, axis=1), jnp.concatenate(tis, axis=1)


def _stage_a_kernel(nchunk, v, k, logits_ref, cum_ref, sc_ref, fi_ref,
                    m_ref, s_ref, tv_ref, ti_ref):
    c = pl.program_id(0)

    @pl.when(c == 0)
    def _init():
        m_ref[...] = jnp.full(m_ref.shape, _NEG_INF, jnp.float32)
        s_ref[...] = jnp.zeros(s_ref.shape, jnp.float32)
        tv_ref[...] = jnp.full(tv_ref.shape, _NEG_INF, jnp.float32)
        ti_ref[...] = jnp.zeros(ti_ref.shape, jnp.int32)

    x = logits_ref[...]
    lane = jax.lax.broadcasted_iota(jnp.int32, x.shape, 1)
    gidx = c * _CHUNK + lane
    x = jnp.where(gidx < v, x, _NEG_INF)

    cmax = jnp.max(x, axis=1, keepdims=True)
    m_old = m_ref[...]
    m_new = jnp.maximum(m_old, cmax)
    s_ref[...] = (s_ref[...] * jnp.exp(m_old - m_new)
                  + jnp.sum(jnp.exp(x - m_new), axis=1, keepdims=True))
    m_ref[...] = m_new

    rowk = jax.lax.broadcasted_iota(jnp.int32, x.shape, 0) % k
    flat = rowk * v + gidx
    ctv, cti = _top8(x, flat)
    mtv, mti = _top8(jnp.concatenate([tv_ref[...], ctv], axis=1),
                     jnp.concatenate([ti_ref[...], cti], axis=1))
    tv_ref[...] = mtv
    ti_ref[...] = mti

    @pl.when(c == nchunk - 1)
    def _fin():
        lse = m_ref[...] + jnp.log(s_ref[...])
        sc_ref[...] = tv_ref[...] + (cum_ref[...] - lse)
        fi_ref[...] = ti_ref[...]


def _stage_b_kernel(sc_ref, fi_ref, tp_ref, ti_ref):
    tv, ti = _top8(sc_ref[...], fi_ref[...])
    tp_ref[...] = tv
    ti_ref[...] = ti


def _stage_c_kernel(v, k, fi_ref, beams_ref, out_ref):
    fi = fi_ref[...]
    rows = beams_ref.shape[0]
    l = beams_ref.shape[1]
    src_k = jnp.floor(fi.astype(jnp.float32) * (1.0 / v)).astype(jnp.int32)
    tok = fi - src_k * v
    r = jax.lax.broadcasted_iota(jnp.int32, (rows, rows), 0)
    ccol = jax.lax.broadcasted_iota(jnp.int32, (rows, rows), 1)
    p = ((r // k) == (ccol // k)) & ((ccol % k) == src_k)
    g = jax.lax.dot(p.astype(jnp.float32), beams_ref[...].astype(jnp.float32),
                    precision=jax.lax.Precision.HIGHEST,
                    preferred_element_type=jnp.float32)
    out_ref[:, :l] = g.astype(jnp.int32)
    out_ref[:, l:l + 1] = tok


def kernel(logits, cumulative_ps, ongoing_beams):
    b, k, v = logits.shape
    l = ongoing_beams.shape[-1]
    rows = b * k
    nchunk = (v + _CHUNK - 1) // _CHUNK

    logits2d = logits.reshape(rows, v)
    cum2d = cumulative_ps.reshape(rows, 1)

    sc, fi = pl.pallas_call(
        functools.partial(_stage_a_kernel, nchunk, v, k),
        grid=(nchunk,),
        in_specs=[pl.BlockSpec((rows, _CHUNK), lambda c: (0, c)),
                  pl.BlockSpec((rows, 1), lambda c: (0, 0))],
        out_specs=[pl.BlockSpec((rows, 8), lambda c: (0, 0)),
                   pl.BlockSpec((rows, 8), lambda c: (0, 0))],
        out_shape=[jax.ShapeDtypeStruct((rows, 8), jnp.float32),
                   jax.ShapeDtypeStruct((rows, 8), jnp.int32)],
        scratch_shapes=[pltpu.VMEM((rows, 1), jnp.float32),
                        pltpu.VMEM((rows, 1), jnp.float32),
                        pltpu.VMEM((rows, 8), jnp.float32),
                        pltpu.VMEM((rows, 8), jnp.int32)],
    )(logits2d, cum2d)

    top_p, ti = pl.pallas_call(
        _stage_b_kernel,
        out_shape=[jax.ShapeDtypeStruct((b, k), jnp.float32),
                   jax.ShapeDtypeStruct((b, k), jnp.int32)],
    )(sc.reshape(b, k * 8), fi.reshape(b, k * 8))

    beams2d = ongoing_beams.reshape(rows, l).astype(jnp.int32)
    nb = pl.pallas_call(
        functools.partial(_stage_c_kernel, v, k),
        out_shape=jax.ShapeDtypeStruct((rows, l + 1), jnp.int32),
    )(ti.reshape(rows, 1), beams2d)

    new_beams = nb.reshape(b, k, l + 1).astype(ongoing_beams.dtype)
    return top_p, new_beams


# speedup vs baseline: 8.6928x; 1.0648x over previous
import functools

import jax
import jax.numpy as jnp
from jax import lax
from jax.experimental import pallas as pl
from jax.experimental.pallas import tpu as pltpu
from jax.experimental.pallas import tpu_sc as plsc

_CHUNK = 8192
_FOLD = 128
_NEG_INF = float("-inf")
_NC, _NS, _LANES = 2, 16, 16


def _top8(v, idx):
    big = jnp.int32(2147483647)
    tvs, tis = [], []
    for _ in range(8):
        m = jnp.max(v, axis=1, keepdims=True)
        i = jnp.min(jnp.where(v == m, idx, big), axis=1, keepdims=True)
        tvs.append(m)
        tis.append(i)
        v = jnp.where(idx == i, _NEG_INF, v)
    return jnp.concatenate(tvs, axis=1), jnp.concatenate(tis, axis=1)


def _stage_a_kernel(nchunk, v, logits_ref, cum_ref, comp_ref, adj_ref,
                    m_ref, s_ref):
    c = pl.program_id(0)

    @pl.when(c == 0)
    def _init():
        m_ref[...] = jnp.full(m_ref.shape, _NEG_INF, jnp.float32)
        s_ref[...] = jnp.zeros(s_ref.shape, jnp.float32)

    x = logits_ref[...]
    lane = jax.lax.broadcasted_iota(jnp.int32, x.shape, 1)
    x = jnp.where(lane < v - c * _CHUNK, x, _NEG_INF)

    f = x
    w = _CHUNK
    while w > _FOLD:
        w //= 2
        f = jnp.maximum(f[:, :w], f[:, w:2 * w])
    comp_ref[...] = f

    cmax = jnp.max(f, axis=1, keepdims=True)
    m_old = m_ref[...]
    m_new = jnp.maximum(m_old, cmax)
    s_new = (s_ref[...] * jnp.exp(m_old - m_new)
             + jnp.sum(jnp.exp(x - m_new), axis=1, keepdims=True))
    s_ref[...] = s_new
    m_ref[...] = m_new

    @pl.when(c == nchunk - 1)
    def _fin():
        adj_ref[...] = cum_ref[...] - (m_new + jnp.log(s_new))


def _merge_top16(rv, ri, bv, bi):
    bv_s, bi_s = plsc.sort_key_val(bv, bi, descending=True)
    bva = lax.rev(bv_s, (0,))
    bia = lax.rev(bi_s, (0,))
    take = rv >= bva
    mv = jnp.where(take, rv, bva)
    mi = jnp.where(take, ri, bia)
    out = plsc.sort_key_val(mv, mi, descending=True)
    return out[0], out[1]


def _make_sc_refine(rows, v, nslots):
    nrows_tbl = rows * v // _LANES
    mesh = plsc.VectorSubcoreMesh(core_axis_name="c", subcore_axis_name="s")
    rpw = rows // (_NC * _NS)

    @functools.partial(
        pl.kernel, mesh=mesh,
        compiler_params=pltpu.CompilerParams(needs_layout_passes=False,
                                             use_tc_tiling_on_sc=False),
        out_type=[jax.ShapeDtypeStruct((rows, 16), jnp.float32),
                  jax.ShapeDtypeStruct((rows, 16), jnp.int32)],
        scratch_types=[
            pltpu.VMEM((rpw, nslots), jnp.float32),
            pltpu.VMEM((8, 128), jnp.int32),
            pltpu.VMEM((1024, 16), jnp.float32),
            pltpu.VMEM((rpw, 16), jnp.float32),
            pltpu.VMEM((rpw, 16), jnp.int32),
            pltpu.SemaphoreType.DMA,
        ],
    )
    def sc_refine(comp_hbm, tbl_hbm, outv_hbm, outi_hbm,
                  comp_v, idx_v, stage_v, ov_v, oi_v, sem):
        wid = lax.axis_index("s") * _NC + lax.axis_index("c")
        base = wid * rpw
        pltpu.sync_copy(comp_hbm.at[pl.ds(base, rpw)], comp_v)
        iota = lax.iota(jnp.int32, _LANES)

        for k in range(rpw):
            r = base + k

            def scan_body(j, carry):
                rv, ri = carry
                bv = comp_v[k, pl.ds(j * 16, 16)]
                bi = j * 16 + iota
                return _merge_top16(rv, ri, bv, bi)

            rv0 = jnp.full((16,), _NEG_INF, jnp.float32)
            ri0 = jnp.zeros((16,), jnp.int32)
            rv, ri = lax.fori_loop(0, nslots // 16, scan_body, (rv0, ri0))

            sids = []
            for s in range(16):
                sid = jnp.sum(jnp.where(iota == s, ri, 0))
                sids.append(sid)
                cc = sid // _FOLD
                l = sid % _FOLD
                row0 = (r * v + cc * _CHUNK + l) // 16
                for t in range(4):
                    vec = row0 + 8 * (16 * t + iota)
                    vec = jnp.minimum(vec, nrows_tbl - 1)
                    idx_v[s // 2, pl.ds(64 * (s % 2) + 16 * t, 16)] = vec

            copies = [
                pltpu.async_copy(tbl_hbm.at[idx_v.at[g]],
                                 stage_v.at[pl.ds(128 * g, 128)], sem)
                for g in range(8)
            ]
            for cp in copies:
                cp.wait()

            rv2 = jnp.full((16,), _NEG_INF, jnp.float32)
            ri2 = jnp.zeros((16,), jnp.int32)
            for s in range(16):
                sid = sids[s]
                cc = sid // _FOLD
                l = sid % _FOLD
                lane_s = l % 16
                for t in range(4):
                    rowidx = 64 * s + 16 * t + iota
                    laneidx = jnp.zeros((16,), jnp.int32) + lane_s
                    vals = plsc.load_gather(stage_v, [rowidx, laneidx])
                    gidx = cc * _CHUNK + l + 128 * (16 * t + iota)
                    vals = jnp.where(gidx < v, vals, _NEG_INF)
                    rv2, ri2 = _merge_top16(rv2, ri2, vals, gidx)

            ov_v[k, :] = rv2
            oi_v[k, :] = ri2

        pltpu.sync_copy(ov_v, outv_hbm.at[pl.ds(base, rpw)])
        pltpu.sync_copy(oi_v, outi_hbm.at[pl.ds(base, rpw)])

    return sc_refine


def _stage_b_kernel(v, cv_ref, ci_ref, adj_ref, tp_ref, ti_ref):
    sc = cv_ref[...] + adj_ref[...]
    lanek = jax.lax.broadcasted_iota(jnp.int32, sc.shape, 1) // 16
    flat = lanek * v + ci_ref[...]
    tv, ti = _top8(sc, flat)
    tp_ref[...] = tv
    ti_ref[...] = ti


def _stage_c_kernel(v, k, fi_ref, beams_ref, out_ref):
    fi = fi_ref[...]
    rows = beams_ref.shape[0]
    l = beams_ref.shape[1]
    src_k = jnp.floor(fi.astype(jnp.float32) * (1.0 / v)).astype(jnp.int32)
    tok = fi - src_k * v
    r = jax.lax.broadcasted_iota(jnp.int32, (rows, rows), 0)
    ccol = jax.lax.broadcasted_iota(jnp.int32, (rows, rows), 1)
    p = ((r // k) == (ccol // k)) & ((ccol % k) == src_k)
    g = jax.lax.dot(p.astype(jnp.float32), beams_ref[...].astype(jnp.float32),
                    precision=jax.lax.Precision.HIGHEST,
                    preferred_element_type=jnp.float32)
    out_ref[:, :l] = g.astype(jnp.int32)
    out_ref[:, l:l + 1] = tok


def kernel(logits, cumulative_ps, ongoing_beams):
    b, k, v = logits.shape
    l = ongoing_beams.shape[-1]
    rows = b * k
    nchunk = (v + _CHUNK - 1) // _CHUNK
    nslots = nchunk * _FOLD

    logits2d = logits.reshape(rows, v)
    cum2d = cumulative_ps.reshape(rows, 1)

    comp, adj = pl.pallas_call(
        functools.partial(_stage_a_kernel, nchunk, v),
        grid=(nchunk,),
        in_specs=[pl.BlockSpec((rows, _CHUNK), lambda c: (0, c)),
                  pl.BlockSpec((rows, 1), lambda c: (0, 0))],
        out_specs=[pl.BlockSpec((rows, _FOLD), lambda c: (0, c)),
                   pl.BlockSpec((rows, 1), lambda c: (0, 0))],
        out_shape=[jax.ShapeDtypeStruct((rows, nslots), jnp.float32),
                   jax.ShapeDtypeStruct((rows, 1), jnp.float32)],
        scratch_shapes=[pltpu.VMEM((rows, 1), jnp.float32),
                        pltpu.VMEM((rows, 1), jnp.float32)],
    )(logits2d, cum2d)

    tbl = logits2d.reshape(rows * v // _LANES, _LANES)
    cands_v, cands_i = _make_sc_refine(rows, v, nslots)(comp, tbl)

    adjr = jnp.repeat(adj.reshape(b, k), 16, axis=1)
    top_p, ti = pl.pallas_call(
        functools.partial(_stage_b_kernel, v),
        out_shape=[jax.ShapeDtypeStruct((b, k), jnp.float32),
                   jax.ShapeDtypeStruct((b, k), jnp.int32)],
    )(cands_v.reshape(b, k * 16), cands_i.reshape(b, k * 16), adjr)

    beams2d = ongoing_beams.reshape(rows, l).astype(jnp.int32)
    nb = pl.pallas_call(
        functools.partial(_stage_c_kernel, v, k),
        out_shape=jax.ShapeDtypeStruct((rows, l + 1), jnp.int32),
    )(ti.reshape(rows, 1), beams2d)

    new_beams = nb.reshape(b, k, l + 1).astype(ongoing_beams.dtype)
    return top_p, new_beams
